# baseline (speedup 1.0000x reference)
import functools

import jax
import jax.numpy as jnp
from jax import lax
from jax.experimental import pallas as pl
from jax.experimental.pallas import tpu as pltpu
from jax.experimental.pallas import tpu_sc as plsc

N_T = 10000
N_P = 5000
N_PAD = 10240
E_REAL = 320000
E_PAD = 327680
NW = 32
CH = 80
EC = 128
G_PAD = 12288
C = 128
ROWS_PER_TILE = N_PAD // 16

_mesh = plsc.VectorSubcoreMesh(core_axis_name="c", subcore_axis_name="s")


@functools.partial(
    pl.kernel,
    mesh=_mesh,
    out_type=[
        jax.ShapeDtypeStruct((G_PAD, C), jnp.float32),
        jax.ShapeDtypeStruct((N_PAD,), jnp.float32),
        jax.ShapeDtypeStruct((N_PAD,), jnp.float32),
    ],
    scratch_types=[
        pltpu.VMEM((3, EC), jnp.int32),
        pltpu.VMEM((CH, EC), jnp.int32),
        pltpu.VMEM((EC, C), jnp.float32),
        pltpu.VMEM((EC, C), jnp.float32),
        pltpu.VMEM((EC, C), jnp.float32),
        pltpu.VMEM((EC,), jnp.float32),
        pltpu.VMEM_SHARED((N_PAD,), jnp.float32),
        pltpu.SemaphoreType.DMA,
        pltpu.SemaphoreType.DMA,
    ],
)
def _sc_prep(x1, idxh3, src3, zeros1, h_out, deg0, deg1,
             idx_v, src_v, hb0, hb1, hb2, ones_v, dacc, sem_h, sem_d):
    HB = (hb0, hb1, hb2)
    cid = lax.axis_index("c")
    sid = lax.axis_index("s")
    wid = sid * 2 + cid
    sl = pl.ds(sid * ROWS_PER_TILE, ROWS_PER_TILE)

    pltpu.sync_copy(zeros1.at[sl], dacc.at[sl])
    pltpu.sync_copy(idxh3.at[wid], idx_v)
    pltpu.sync_copy(src3.at[wid], src_v)
    for i in range(EC // 16):
        ones_v[pl.ds(i * 16, 16)] = jnp.ones((16,), jnp.float32)
    plsc.subcore_barrier()

    h_copies = [pltpu.async_copy(x1.at[idx_v.at[t]], HB[t], sem_h)
                for t in range(3)]

    def deg_body(r, carry):
        copies = [pltpu.async_copy(ones_v, dacc.at[src_v.at[r * 8 + b]],
                                   sem_d, add=True)
                  for b in range(8)]
        for cp in copies:
            cp.wait()
        return carry
    lax.fori_loop(0, CH // 8, deg_body, 0)

    for t in range(3):
        h_copies[t].wait()
        pltpu.sync_copy(HB[t], h_out.at[pl.ds(wid * 384 + t * EC, EC)])

    plsc.subcore_barrier()

    @pl.when(cid == 0)
    def _():
        pltpu.sync_copy(dacc.at[sl], deg0.at[sl])

    @pl.when(cid == 1)
    def _():
        pltpu.sync_copy(dacc.at[sl], deg1.at[sl])


@functools.partial(
    pl.kernel,
    mesh=_mesh,
    out_type=jax.ShapeDtypeStruct((2, N_PAD, C), jnp.float32),
    scratch_types=[
        pltpu.VMEM((16, EC), jnp.int32),
        pltpu.VMEM((16, EC), jnp.int32),
        pltpu.VMEM((EC, C), jnp.float32),
        pltpu.VMEM((EC, C), jnp.float32),
        pltpu.VMEM_SHARED((N_PAD, C), jnp.float32),
        pltpu.SemaphoreType.DMA,
        pltpu.SemaphoreType.DMA,
    ],
)
def _sc_prop(v_hbm, src3, dst3, zeros2, agg_out, src_v, dst_v,
             rb0, rb1, acc, sg0, sg1):
    cid = lax.axis_index("c")
    sid = lax.axis_index("s")
    wid = sid * 2 + cid
    sl = pl.ds(sid * ROWS_PER_TILE, ROWS_PER_TILE)

    pltpu.sync_copy(zeros2.at[sl], acc.at[sl])
    plsc.subcore_barrier()

    def outer(blk, carry):
        base = blk * 16
        pltpu.sync_copy(src3.at[wid, pl.ds(base, 16)], src_v)
        pltpu.sync_copy(dst3.at[wid, pl.ds(base, 16)], dst_v)
        pltpu.async_copy(v_hbm.at[src_v.at[0]], rb0, sg0)

        def inner(r, c2):
            a = r * 2
            pltpu.make_async_copy(v_hbm.at[src_v.at[a]], rb0, sg0).wait()
            pltpu.async_copy(v_hbm.at[src_v.at[a + 1]], rb1, sg1)
            pltpu.sync_copy(rb0, acc.at[dst_v.at[a]], add=True)
            pltpu.make_async_copy(v_hbm.at[src_v.at[a + 1]], rb1,
                                  sg1).wait()

            @pl.when(r < 7)
            def _():
                pltpu.async_copy(v_hbm.at[src_v.at[a + 2]], rb0, sg0)
            pltpu.sync_copy(rb1, acc.at[dst_v.at[a + 1]], add=True)
            return c2
        lax.fori_loop(0, 8, inner, 0)
        return carry
    lax.fori_loop(0, CH // 16, outer, 0)

    plsc.subcore_barrier()
    pltpu.sync_copy(acc.at[sl], agg_out.at[cid, sl])


def _dis_body(d0_ref, d1_ref, dis_ref):
    d = d0_ref[...] + d1_ref[...]
    r = lax.broadcasted_iota(jnp.int32, (80, C), 0)
    c = lax.broadcasted_iota(jnp.int32, (80, C), 1)
    ok = (d > 0.0) & (r * C + c < N_T)
    dis_ref[...] = jnp.where(ok, lax.rsqrt(jnp.maximum(d, 1e-30)), 0.0)


def _tc_dis(deg0, deg1):
    return pl.pallas_call(
        _dis_body,
        out_shape=jax.ShapeDtypeStruct((80, C), jnp.float32),
    )(deg0.reshape(80, C), deg1.reshape(80, C))


_BLK = 1024
_GRID = N_PAD // _BLK


def _scale_body(x_ref, dis_ref, v_ref):
    v_ref[...] = dis_ref[...] * x_ref[...]


def _tc_scale(x, dis2):
    return pl.pallas_call(
        _scale_body,
        grid=(_GRID,),
        in_specs=[pl.BlockSpec((_BLK, C), lambda i: (i, 0)),
                  pl.BlockSpec((_BLK, C), lambda i: (i, 0))],
        out_specs=pl.BlockSpec((_BLK, C), lambda i: (i, 0)),
        out_shape=jax.ShapeDtypeStruct((N_PAD, C), jnp.float32),
    )(x, dis2)


def _make_txv(coef, use_prev, emit_v):
    def body(*refs):
        it = iter(refs)
        agg_ref = next(it)
        dis_ref = next(it)
        prev_ref = next(it) if use_prev else None
        tx_ref = next(it)
        v_ref = next(it) if emit_v else None

        dis = dis_ref[...]
        t = coef * (dis * (agg_ref[0] + agg_ref[1]))
        if use_prev:
            t = t - prev_ref[...]
        tx_ref[...] = t
        if emit_v:
            v_ref[...] = dis * t
    return body


def _tc_txv(agg, dis2, prev, coef, emit_v):
    use_prev = prev is not None
    in_specs = [pl.BlockSpec((2, _BLK, C), lambda i: (0, i, 0)),
                pl.BlockSpec((_BLK, C), lambda i: (i, 0))]
    args = [agg, dis2]
    if use_prev:
        in_specs.append(pl.BlockSpec((_BLK, C), lambda i: (i, 0)))
        args.append(prev)
    n_out = 2 if emit_v else 1
    res = pl.pallas_call(
        _make_txv(coef, use_prev, emit_v),
        grid=(_GRID,),
        in_specs=in_specs,
        out_specs=[pl.BlockSpec((_BLK, C), lambda i: (i, 0))] * n_out,
        out_shape=[jax.ShapeDtypeStruct((N_PAD, C), jnp.float32)] * n_out,
    )(*args)
    return (res[0], res[1]) if emit_v else (res[0], None)


def _make_mm(use_acc, add_bias):
    def body(*refs):
        it = iter(refs)
        acc_ref = next(it) if use_acc else None
        x_ref = next(it)
        w_ref = next(it)
        b_ref = next(it) if add_bias else None
        out_ref = next(it)

        o = jnp.dot(x_ref[...], w_ref[...],
                    preferred_element_type=jnp.float32)
        if use_acc:
            o = o + acc_ref[...]
        if add_bias:
            o = o + b_ref[...]
        out_ref[...] = o
    return body


def _tc_mm(acc, x, w, b):
    use_acc = acc is not None
    add_bias = b is not None
    in_specs = []
    args = []
    if use_acc:
        in_specs.append(pl.BlockSpec((_BLK, C), lambda i: (i, 0)))
        args.append(acc)
    in_specs.append(pl.BlockSpec((_BLK, C), lambda i: (i, 0)))
    args.append(x)
    in_specs.append(pl.BlockSpec((C, C), lambda i: (0, 0)))
    args.append(w)
    if add_bias:
        in_specs.append(pl.BlockSpec((1, C), lambda i: (0, 0)))
        args.append(b.reshape(1, C))
    aliases = {0: 0} if use_acc else {}
    return pl.pallas_call(
        _make_mm(use_acc, add_bias),
        grid=(_GRID,),
        in_specs=in_specs,
        out_specs=pl.BlockSpec((_BLK, C), lambda i: (i, 0)),
        out_shape=jax.ShapeDtypeStruct((N_PAD, C), jnp.float32),
        input_output_aliases=aliases,
    )(*args)


def kernel(x, perm, edge_index, W1, b1, W2, b2, Wl, bl):
    t = jnp.arange(N_P, dtype=jnp.int32)
    tlast = jnp.full((N_T,), -1, jnp.int32).at[perm].max(t)
    p_hi = jnp.arange(N_P, N_T, dtype=jnp.int32)
    pv_low = jnp.where(tlast[:N_P] > t, tlast[:N_P], perm)
    pv_high = jnp.where(tlast[N_P:] >= 0, tlast[N_P:], p_hi)
    pv = jnp.concatenate([pv_low, pv_high])

    spread = jnp.arange(N_T, dtype=jnp.int32) % 240
    idxh = jnp.where(pv < N_P, pv, N_P + spread)
    pad_idx = N_P + (jnp.arange(G_PAD - N_T, dtype=jnp.int32) % 240)
    idxh3 = jnp.concatenate([idxh, pad_idx]).reshape(NW, 3, EC)
    x1 = jnp.concatenate([x, jnp.zeros((240, C), jnp.float32)], axis=0)

    n_dummy = E_PAD - E_REAL
    dpad = N_T + (jnp.arange(n_dummy, dtype=jnp.int32) % 240)
    src3 = jnp.concatenate([edge_index[0], dpad]).reshape(NW, CH, EC)
    dst3 = jnp.concatenate([edge_index[1], dpad]).reshape(NW, CH, EC)

    zeros1 = jnp.zeros((N_PAD,), jnp.float32)
    zeros2 = jnp.zeros((N_PAD, C), jnp.float32)

    h_g, deg0, deg1 = _sc_prep(x1, idxh3, src3, zeros1)
    h = h_g[:N_PAD]

    dis = _tc_dis(deg0, deg1)
    dis2 = jnp.broadcast_to(dis.reshape(N_PAD, 1), (N_PAD, C))

    state = h
    for (W, b) in ((W1, b1), (W2, b2)):
        v = _tc_scale(state, dis2)
        acc = _tc_mm(None, state, W[0], None)
        txs = [state]
        for k in range(1, 6):
            agg = _sc_prop(v, src3, dst3, zeros2)
            coef = -1.0 if k == 1 else -2.0
            prev = None if k == 1 else txs[k - 2]
            tx, v = _tc_txv(agg, dis2, prev, coef, emit_v=(k < 5))
            acc = _tc_mm(acc, tx, W[k], b if k == 5 else None)
            txs.append(tx)
        state = acc

    out = _tc_mm(None, state, Wl, bl)
    return out[:N_T]


# speedup vs baseline: 1.0156x; 1.0156x over previous
import functools

import jax
import jax.numpy as jnp
from jax import lax
from jax.experimental import pallas as pl
from jax.experimental.pallas import tpu as pltpu
from jax.experimental.pallas import tpu_sc as plsc

N_T = 10000
N_P = 5000
N_PAD = 10240
E_REAL = 320000
E_PAD = 327680
NW = 32
CH = 80
EC = 128
G_PAD = 12288
C = 128
ROWS_PER_TILE = N_PAD // 16

_mesh = plsc.VectorSubcoreMesh(core_axis_name="c", subcore_axis_name="s")


@functools.partial(
    pl.kernel,
    mesh=_mesh,
    out_type=[
        jax.ShapeDtypeStruct((G_PAD, C), jnp.float32),
        jax.ShapeDtypeStruct((N_PAD,), jnp.float32),
        jax.ShapeDtypeStruct((N_PAD,), jnp.float32),
    ],
    scratch_types=[
        pltpu.VMEM((3, EC), jnp.int32),
        pltpu.VMEM((CH, EC), jnp.int32),
        pltpu.VMEM((EC, C), jnp.float32),
        pltpu.VMEM((EC, C), jnp.float32),
        pltpu.VMEM((EC, C), jnp.float32),
        pltpu.VMEM((EC,), jnp.float32),
        pltpu.VMEM_SHARED((N_PAD,), jnp.float32),
        pltpu.SemaphoreType.DMA,
        pltpu.SemaphoreType.DMA,
    ],
)
def _sc_prep(x1, idxh3, src3, zeros1, h_out, deg0, deg1,
             idx_v, src_v, hb0, hb1, hb2, ones_v, dacc, sem_h, sem_d):
    HB = (hb0, hb1, hb2)
    cid = lax.axis_index("c")
    sid = lax.axis_index("s")
    wid = sid * 2 + cid
    sl = pl.ds(sid * ROWS_PER_TILE, ROWS_PER_TILE)

    pltpu.sync_copy(zeros1.at[sl], dacc.at[sl])
    pltpu.sync_copy(idxh3.at[wid], idx_v)
    pltpu.sync_copy(src3.at[wid], src_v)
    for i in range(EC // 16):
        ones_v[pl.ds(i * 16, 16)] = jnp.ones((16,), jnp.float32)
    plsc.subcore_barrier()

    h_copies = [pltpu.async_copy(x1.at[idx_v.at[t]], HB[t], sem_h)
                for t in range(3)]

    def deg_body(r, carry):
        copies = [pltpu.async_copy(ones_v, dacc.at[src_v.at[r * 8 + b]],
                                   sem_d, add=True)
                  for b in range(8)]
        for cp in copies:
            cp.wait()
        return carry
    lax.fori_loop(0, CH // 8, deg_body, 0)

    for t in range(3):
        h_copies[t].wait()
        pltpu.sync_copy(HB[t], h_out.at[pl.ds(wid * 384 + t * EC, EC)])

    plsc.subcore_barrier()

    @pl.when(cid == 0)
    def _():
        pltpu.sync_copy(dacc.at[sl], deg0.at[sl])

    @pl.when(cid == 1)
    def _():
        pltpu.sync_copy(dacc.at[sl], deg1.at[sl])


@functools.partial(
    pl.kernel,
    mesh=_mesh,
    out_type=jax.ShapeDtypeStruct((2, N_PAD, C), jnp.float32),
    scratch_types=[
        pltpu.VMEM((16, EC), jnp.int32),
        pltpu.VMEM((16, EC), jnp.int32),
        pltpu.VMEM((16, EC), jnp.int32),
        pltpu.VMEM((16, EC), jnp.int32),
        pltpu.VMEM((EC, C), jnp.float32),
        pltpu.VMEM((EC, C), jnp.float32),
        pltpu.VMEM_SHARED((N_PAD, C), jnp.float32),
        pltpu.SemaphoreType.DMA,
        pltpu.SemaphoreType.DMA,
        pltpu.SemaphoreType.DMA,
    ],
)
def _sc_prop(v_hbm, src3, dst3, zeros2, agg_out, src_va, dst_va,
             src_vb, dst_vb, rb0, rb1, acc, sg0, sg1, si):
    cid = lax.axis_index("c")
    sid = lax.axis_index("s")
    wid = sid * 2 + cid
    sl = pl.ds(sid * ROWS_PER_TILE, ROWS_PER_TILE)

    pltpu.sync_copy(zeros2.at[sl], acc.at[sl])
    plsc.subcore_barrier()

    NBLK = CH // 16
    pltpu.sync_copy(src3.at[wid, pl.ds(0, 16)], src_va)
    pltpu.sync_copy(dst3.at[wid, pl.ds(0, 16)], dst_va)
    for blk in range(NBLK):
        src_v, dst_v = (src_va, dst_va) if blk % 2 == 0 else (src_vb,
                                                              dst_vb)
        idx_cp = None
        if blk + 1 < NBLK:
            nsv, ndv = (src_vb, dst_vb) if blk % 2 == 0 else (src_va,
                                                              dst_va)
            base = (blk + 1) * 16
            pltpu.async_copy(src3.at[wid, pl.ds(base, 16)], nsv, si)
            idx_cp = pltpu.async_copy(dst3.at[wid, pl.ds(base, 16)],
                                      ndv, si)
        pltpu.async_copy(v_hbm.at[src_v.at[0]], rb0, sg0)

        def inner(r, c2, src_v=src_v, dst_v=dst_v):
            a = r * 2
            pltpu.make_async_copy(v_hbm.at[src_v.at[a]], rb0, sg0).wait()
            pltpu.async_copy(v_hbm.at[src_v.at[a + 1]], rb1, sg1)
            pltpu.sync_copy(rb0, acc.at[dst_v.at[a]], add=True)
            pltpu.make_async_copy(v_hbm.at[src_v.at[a + 1]], rb1,
                                  sg1).wait()

            @pl.when(r < 7)
            def _():
                pltpu.async_copy(v_hbm.at[src_v.at[a + 2]], rb0, sg0)
            pltpu.sync_copy(rb1, acc.at[dst_v.at[a + 1]], add=True)
            return c2
        lax.fori_loop(0, 8, inner, 0)
        if idx_cp is not None:
            pltpu.make_async_copy(src3.at[wid, pl.ds(0, 16)], src_vb,
                                  si).wait()
            pltpu.make_async_copy(src3.at[wid, pl.ds(0, 16)], dst_vb,
                                  si).wait()

    plsc.subcore_barrier()
    pltpu.sync_copy(acc.at[sl], agg_out.at[cid, sl])


def _dis_body(d0_ref, d1_ref, dis_ref):
    d = d0_ref[...] + d1_ref[...]
    r = lax.broadcasted_iota(jnp.int32, (80, C), 0)
    c = lax.broadcasted_iota(jnp.int32, (80, C), 1)
    ok = (d > 0.0) & (r * C + c < N_T)
    dis_ref[...] = jnp.where(ok, lax.rsqrt(jnp.maximum(d, 1e-30)), 0.0)


def _tc_dis(deg0, deg1):
    return pl.pallas_call(
        _dis_body,
        out_shape=jax.ShapeDtypeStruct((80, C), jnp.float32),
    )(deg0.reshape(80, C), deg1.reshape(80, C))


_BLK = 1024
_GRID = N_PAD // _BLK


def _scale_body(x_ref, dis_ref, v_ref):
    v_ref[...] = dis_ref[...] * x_ref[...]


def _tc_scale(x, dis2):
    return pl.pallas_call(
        _scale_body,
        grid=(_GRID,),
        in_specs=[pl.BlockSpec((_BLK, C), lambda i: (i, 0)),
                  pl.BlockSpec((_BLK, C), lambda i: (i, 0))],
        out_specs=pl.BlockSpec((_BLK, C), lambda i: (i, 0)),
        out_shape=jax.ShapeDtypeStruct((N_PAD, C), jnp.float32),
    )(x, dis2)


def _make_txv(coef, use_prev, emit_v):
    def body(*refs):
        it = iter(refs)
        agg_ref = next(it)
        dis_ref = next(it)
        prev_ref = next(it) if use_prev else None
        tx_ref = next(it)
        v_ref = next(it) if emit_v else None

        dis = dis_ref[...]
        t = coef * (dis * (agg_ref[0] + agg_ref[1]))
        if use_prev:
            t = t - prev_ref[...]
        tx_ref[...] = t
        if emit_v:
            v_ref[...] = dis * t
    return body


def _tc_txv(agg, dis2, prev, coef, emit_v):
    use_prev = prev is not None
    in_specs = [pl.BlockSpec((2, _BLK, C), lambda i: (0, i, 0)),
                pl.BlockSpec((_BLK, C), lambda i: (i, 0))]
    args = [agg, dis2]
    if use_prev:
        in_specs.append(pl.BlockSpec((_BLK, C), lambda i: (i, 0)))
        args.append(prev)
    n_out = 2 if emit_v else 1
    res = pl.pallas_call(
        _make_txv(coef, use_prev, emit_v),
        grid=(_GRID,),
        in_specs=in_specs,
        out_specs=[pl.BlockSpec((_BLK, C), lambda i: (i, 0))] * n_out,
        out_shape=[jax.ShapeDtypeStruct((N_PAD, C), jnp.float32)] * n_out,
    )(*args)
    return (res[0], res[1]) if emit_v else (res[0], None)


def _make_mm(use_acc, add_bias):
    def body(*refs):
        it = iter(refs)
        acc_ref = next(it) if use_acc else None
        x_ref = next(it)
        w_ref = next(it)
        b_ref = next(it) if add_bias else None
        out_ref = next(it)

        o = jnp.dot(x_ref[...], w_ref[...],
                    preferred_element_type=jnp.float32)
        if use_acc:
            o = o + acc_ref[...]
        if add_bias:
            o = o + b_ref[...]
        out_ref[...] = o
    return body


def _tc_mm(acc, x, w, b):
    use_acc = acc is not None
    add_bias = b is not None
    in_specs = []
    args = []
    if use_acc:
        in_specs.append(pl.BlockSpec((_BLK, C), lambda i: (i, 0)))
        args.append(acc)
    in_specs.append(pl.BlockSpec((_BLK, C), lambda i: (i, 0)))
    args.append(x)
    in_specs.append(pl.BlockSpec((C, C), lambda i: (0, 0)))
    args.append(w)
    if add_bias:
        in_specs.append(pl.BlockSpec((1, C), lambda i: (0, 0)))
        args.append(b.reshape(1, C))
    aliases = {0: 0} if use_acc else {}
    return pl.pallas_call(
        _make_mm(use_acc, add_bias),
        grid=(_GRID,),
        in_specs=in_specs,
        out_specs=pl.BlockSpec((_BLK, C), lambda i: (i, 0)),
        out_shape=jax.ShapeDtypeStruct((N_PAD, C), jnp.float32),
        input_output_aliases=aliases,
    )(*args)


def kernel(x, perm, edge_index, W1, b1, W2, b2, Wl, bl):
    t = jnp.arange(N_P, dtype=jnp.int32)
    tlast = jnp.full((N_T,), -1, jnp.int32).at[perm].max(t)
    p_hi = jnp.arange(N_P, N_T, dtype=jnp.int32)
    pv_low = jnp.where(tlast[:N_P] > t, tlast[:N_P], perm)
    pv_high = jnp.where(tlast[N_P:] >= 0, tlast[N_P:], p_hi)
    pv = jnp.concatenate([pv_low, pv_high])

    spread = jnp.arange(N_T, dtype=jnp.int32) % 240
    idxh = jnp.where(pv < N_P, pv, N_P + spread)
    pad_idx = N_P + (jnp.arange(G_PAD - N_T, dtype=jnp.int32) % 240)
    idxh3 = jnp.concatenate([idxh, pad_idx]).reshape(NW, 3, EC)
    x1 = jnp.concatenate([x, jnp.zeros((240, C), jnp.float32)], axis=0)

    n_dummy = E_PAD - E_REAL
    dpad = N_T + (jnp.arange(n_dummy, dtype=jnp.int32) % 240)
    src3 = jnp.concatenate([edge_index[0], dpad]).reshape(NW, CH, EC)
    dst3 = jnp.concatenate([edge_index[1], dpad]).reshape(NW, CH, EC)

    zeros1 = jnp.zeros((N_PAD,), jnp.float32)
    zeros2 = jnp.zeros((N_PAD, C), jnp.float32)

    h_g, deg0, deg1 = _sc_prep(x1, idxh3, src3, zeros1)
    h = h_g[:N_PAD]

    dis = _tc_dis(deg0, deg1)
    dis2 = jnp.broadcast_to(dis.reshape(N_PAD, 1), (N_PAD, C))

    state = h
    for (W, b) in ((W1, b1), (W2, b2)):
        v = _tc_scale(state, dis2)
        acc = _tc_mm(None, state, W[0], None)
        txs = [state]
        for k in range(1, 6):
            agg = _sc_prop(v, src3, dst3, zeros2)
            coef = -1.0 if k == 1 else -2.0
            prev = None if k == 1 else txs[k - 2]
            tx, v = _tc_txv(agg, dis2, prev, coef, emit_v=(k < 5))
            acc = _tc_mm(acc, tx, W[k], b if k == 5 else None)
            txs.append(tx)
        state = acc

    out = _tc_mm(None, state, Wl, bl)
    return out[:N_T]


# speedup vs baseline: 1.0157x; 1.0001x over previous
import functools

import jax
import jax.numpy as jnp
from jax import lax
from jax.experimental import pallas as pl
from jax.experimental.pallas import tpu as pltpu
from jax.experimental.pallas import tpu_sc as plsc

N_T = 10000
N_P = 5000
N_PAD = 10240
E_REAL = 320000
E_PAD = 327680
NW = 32
CH = 80
EC = 128
G_PAD = 12288
C = 128
ROWS_PER_TILE = N_PAD // 16

_mesh = plsc.VectorSubcoreMesh(core_axis_name="c", subcore_axis_name="s")


@functools.partial(
    pl.kernel,
    mesh=_mesh,
    out_type=[
        jax.ShapeDtypeStruct((G_PAD, C), jnp.float32),
        jax.ShapeDtypeStruct((N_PAD,), jnp.float32),
        jax.ShapeDtypeStruct((N_PAD,), jnp.float32),
    ],
    scratch_types=[
        pltpu.VMEM((3, EC), jnp.int32),
        pltpu.VMEM((CH, EC), jnp.int32),
        pltpu.VMEM((EC, C), jnp.float32),
        pltpu.VMEM((EC, C), jnp.float32),
        pltpu.VMEM((EC, C), jnp.float32),
        pltpu.VMEM((EC,), jnp.float32),
        pltpu.VMEM_SHARED((N_PAD,), jnp.float32),
        pltpu.SemaphoreType.DMA,
        pltpu.SemaphoreType.DMA,
    ],
)
def _sc_prep(x1, idxh3, src3, zeros1, h_out, deg0, deg1,
             idx_v, src_v, hb0, hb1, hb2, ones_v, dacc, sem_h, sem_d):
    HB = (hb0, hb1, hb2)
    cid = lax.axis_index("c")
    sid = lax.axis_index("s")
    wid = sid * 2 + cid
    sl = pl.ds(sid * ROWS_PER_TILE, ROWS_PER_TILE)

    pltpu.sync_copy(zeros1.at[sl], dacc.at[sl])
    pltpu.sync_copy(idxh3.at[wid], idx_v)
    pltpu.sync_copy(src3.at[wid], src_v)
    for i in range(EC // 16):
        ones_v[pl.ds(i * 16, 16)] = jnp.ones((16,), jnp.float32)
    plsc.subcore_barrier()

    h_copies = [pltpu.async_copy(x1.at[idx_v.at[t]], HB[t], sem_h)
                for t in range(3)]

    def deg_body(r, carry):
        copies = [pltpu.async_copy(ones_v, dacc.at[src_v.at[r * 8 + b]],
                                   sem_d, add=True)
                  for b in range(8)]
        for cp in copies:
            cp.wait()
        return carry
    lax.fori_loop(0, CH // 8, deg_body, 0)

    for t in range(3):
        h_copies[t].wait()
        pltpu.sync_copy(HB[t], h_out.at[pl.ds(wid * 384 + t * EC, EC)])

    plsc.subcore_barrier()

    @pl.when(cid == 0)
    def _():
        pltpu.sync_copy(dacc.at[sl], deg0.at[sl])

    @pl.when(cid == 1)
    def _():
        pltpu.sync_copy(dacc.at[sl], deg1.at[sl])


@functools.partial(
    pl.kernel,
    mesh=_mesh,
    out_type=jax.ShapeDtypeStruct((2, N_PAD, C), jnp.float32),
    scratch_types=[
        pltpu.VMEM((16, EC), jnp.int32),
        pltpu.VMEM((16, EC), jnp.int32),
        pltpu.VMEM((16, EC), jnp.int32),
        pltpu.VMEM((16, EC), jnp.int32),
        pltpu.VMEM((EC, C), jnp.float32),
        pltpu.VMEM((EC, C), jnp.float32),
        pltpu.VMEM_SHARED((N_PAD, C), jnp.float32),
        pltpu.SemaphoreType.DMA,
        pltpu.SemaphoreType.DMA,
        pltpu.SemaphoreType.DMA,
    ],
)
def _sc_prop(v_hbm, src3, dst3, zeros2, agg_out, src_va, dst_va,
             src_vb, dst_vb, rb0, rb1, acc, sg0, sg1, si):
    cid = lax.axis_index("c")
    sid = lax.axis_index("s")
    wid = sid * 2 + cid
    sl = pl.ds(sid * ROWS_PER_TILE, ROWS_PER_TILE)

    pltpu.sync_copy(zeros2.at[sl], acc.at[sl])
    plsc.subcore_barrier()

    NBLK = CH // 16
    pltpu.sync_copy(src3.at[wid, pl.ds(0, 16)], src_va)
    pltpu.sync_copy(dst3.at[wid, pl.ds(0, 16)], dst_va)
    for blk in range(NBLK):
        src_v, dst_v = (src_va, dst_va) if blk % 2 == 0 else (src_vb,
                                                              dst_vb)
        idx_cp = None
        if blk + 1 < NBLK:
            nsv, ndv = (src_vb, dst_vb) if blk % 2 == 0 else (src_va,
                                                              dst_va)
            base = (blk + 1) * 16
            pltpu.async_copy(src3.at[wid, pl.ds(base, 16)], nsv, si)
            idx_cp = pltpu.async_copy(dst3.at[wid, pl.ds(base, 16)],
                                      ndv, si)
        pltpu.async_copy(v_hbm.at[src_v.at[0]], rb0, sg0)

        def inner(r, c2, src_v=src_v, dst_v=dst_v):
            a = r * 2
            pltpu.make_async_copy(v_hbm.at[src_v.at[a]], rb0, sg0).wait()
            pltpu.async_copy(v_hbm.at[src_v.at[a + 1]], rb1, sg1)
            pltpu.sync_copy(rb0, acc.at[dst_v.at[a]], add=True)
            pltpu.make_async_copy(v_hbm.at[src_v.at[a + 1]], rb1,
                                  sg1).wait()

            @pl.when(r < 7)
            def _():
                pltpu.async_copy(v_hbm.at[src_v.at[a + 2]], rb0, sg0)
            pltpu.sync_copy(rb1, acc.at[dst_v.at[a + 1]], add=True)
            return c2
        lax.fori_loop(0, 8, inner, 0)
        if idx_cp is not None:
            pltpu.make_async_copy(src3.at[wid, pl.ds(0, 16)], src_vb,
                                  si).wait()
            pltpu.make_async_copy(src3.at[wid, pl.ds(0, 16)], dst_vb,
                                  si).wait()

    plsc.subcore_barrier()
    pltpu.sync_copy(acc.at[sl], agg_out.at[cid, sl])


def _dis_body(d0_ref, d1_ref, dis_ref):
    d = d0_ref[...] + d1_ref[...]
    r = lax.broadcasted_iota(jnp.int32, (80, C), 0)
    c = lax.broadcasted_iota(jnp.int32, (80, C), 1)
    ok = (d > 0.0) & (r * C + c < N_T)
    dis_ref[...] = jnp.where(ok, lax.rsqrt(jnp.maximum(d, 1e-30)), 0.0)


def _tc_dis(deg0, deg1):
    return pl.pallas_call(
        _dis_body,
        out_shape=jax.ShapeDtypeStruct((80, C), jnp.float32),
    )(deg0.reshape(80, C), deg1.reshape(80, C))


_BLK = 1024
_GRID = N_PAD // _BLK


def _scale_body(x_ref, dis_ref, v_ref):
    v_ref[...] = dis_ref[...] * x_ref[...]


def _tc_scale(x, dis2):
    return pl.pallas_call(
        _scale_body,
        grid=(_GRID,),
        in_specs=[pl.BlockSpec((_BLK, C), lambda i: (i, 0)),
                  pl.BlockSpec((_BLK, C), lambda i: (i, 0))],
        out_specs=pl.BlockSpec((_BLK, C), lambda i: (i, 0)),
        out_shape=jax.ShapeDtypeStruct((N_PAD, C), jnp.float32),
    )(x, dis2)


def _make_txv(coef, use_prev, emit_v):
    def body(*refs):
        it = iter(refs)
        agg_ref = next(it)
        dis_ref = next(it)
        prev_ref = next(it) if use_prev else None
        tx_ref = next(it)
        v_ref = next(it) if emit_v else None

        dis = dis_ref[...]
        t = coef * (dis * (agg_ref[0] + agg_ref[1]))
        if use_prev:
            t = t - prev_ref[...]
        tx_ref[...] = t
        if emit_v:
            v_ref[...] = dis * t
    return body


def _tc_txv(agg, dis2, prev, coef, emit_v):
    use_prev = prev is not None
    in_specs = [pl.BlockSpec((2, _BLK, C), lambda i: (0, i, 0)),
                pl.BlockSpec((_BLK, C), lambda i: (i, 0))]
    args = [agg, dis2]
    if use_prev:
        in_specs.append(pl.BlockSpec((_BLK, C), lambda i: (i, 0)))
        args.append(prev)
    n_out = 2 if emit_v else 1
    res = pl.pallas_call(
        _make_txv(coef, use_prev, emit_v),
        grid=(_GRID,),
        in_specs=in_specs,
        out_specs=[pl.BlockSpec((_BLK, C), lambda i: (i, 0))] * n_out,
        out_shape=[jax.ShapeDtypeStruct((N_PAD, C), jnp.float32)] * n_out,
    )(*args)
    return (res[0], res[1]) if emit_v else (res[0], None)


def _make_mm(use_acc, add_bias, emit_v):
    def body(*refs):
        it = iter(refs)
        acc_ref = next(it) if use_acc else None
        x_ref = next(it)
        w_ref = next(it)
        b_ref = next(it) if add_bias else None
        dis_ref = next(it) if emit_v else None
        out_ref = next(it)
        v_ref = next(it) if emit_v else None

        o = jnp.dot(x_ref[...], w_ref[...],
                    preferred_element_type=jnp.float32)
        if use_acc:
            o = o + acc_ref[...]
        if add_bias:
            o = o + b_ref[...]
        out_ref[...] = o
        if emit_v:
            v_ref[...] = dis_ref[...] * o
    return body


def _tc_mm(acc, x, w, b, dis2=None):
    use_acc = acc is not None
    add_bias = b is not None
    emit_v = dis2 is not None
    in_specs = []
    args = []
    if use_acc:
        in_specs.append(pl.BlockSpec((_BLK, C), lambda i: (i, 0)))
        args.append(acc)
    in_specs.append(pl.BlockSpec((_BLK, C), lambda i: (i, 0)))
    args.append(x)
    in_specs.append(pl.BlockSpec((C, C), lambda i: (0, 0)))
    args.append(w)
    if add_bias:
        in_specs.append(pl.BlockSpec((1, C), lambda i: (0, 0)))
        args.append(b.reshape(1, C))
    if emit_v:
        in_specs.append(pl.BlockSpec((_BLK, C), lambda i: (i, 0)))
        args.append(dis2)
    aliases = {0: 0} if use_acc else {}
    n_out = 2 if emit_v else 1
    out_specs = [pl.BlockSpec((_BLK, C), lambda i: (i, 0))] * n_out
    out_shape = [jax.ShapeDtypeStruct((N_PAD, C), jnp.float32)] * n_out
    res = pl.pallas_call(
        _make_mm(use_acc, add_bias, emit_v),
        grid=(_GRID,),
        in_specs=in_specs,
        out_specs=out_specs if emit_v else out_specs[0],
        out_shape=out_shape if emit_v else out_shape[0],
        input_output_aliases=aliases,
    )(*args)
    return res


def kernel(x, perm, edge_index, W1, b1, W2, b2, Wl, bl):
    t = jnp.arange(N_P, dtype=jnp.int32)
    tlast = jnp.full((N_T,), -1, jnp.int32).at[perm].max(t)
    p_hi = jnp.arange(N_P, N_T, dtype=jnp.int32)
    pv_low = jnp.where(tlast[:N_P] > t, tlast[:N_P], perm)
    pv_high = jnp.where(tlast[N_P:] >= 0, tlast[N_P:], p_hi)
    pv = jnp.concatenate([pv_low, pv_high])

    spread = jnp.arange(N_T, dtype=jnp.int32) % 240
    idxh = jnp.where(pv < N_P, pv, N_P + spread)
    pad_idx = N_P + (jnp.arange(G_PAD - N_T, dtype=jnp.int32) % 240)
    idxh3 = jnp.concatenate([idxh, pad_idx]).reshape(NW, 3, EC)
    x1 = jnp.concatenate([x, jnp.zeros((240, C), jnp.float32)], axis=0)

    n_dummy = E_PAD - E_REAL
    dpad = N_T + (jnp.arange(n_dummy, dtype=jnp.int32) % 240)
    src3 = jnp.concatenate([edge_index[0], dpad]).reshape(NW, CH, EC)
    dst3 = jnp.concatenate([edge_index[1], dpad]).reshape(NW, CH, EC)

    zeros1 = jnp.zeros((N_PAD,), jnp.float32)
    zeros2 = jnp.zeros((N_PAD, C), jnp.float32)

    h_g, deg0, deg1 = _sc_prep(x1, idxh3, src3, zeros1)
    h = h_g[:N_PAD]

    dis = _tc_dis(deg0, deg1)
    dis2 = jnp.broadcast_to(dis.reshape(N_PAD, 1), (N_PAD, C))

    state = h
    v = _tc_scale(h, dis2)
    for li, (W, b) in enumerate(((W1, b1), (W2, b2))):
        acc = _tc_mm(None, state, W[0], None)
        txs = [state]
        for k in range(1, 6):
            agg = _sc_prop(v, src3, dst3, zeros2)
            coef = -1.0 if k == 1 else -2.0
            prev = None if k == 1 else txs[k - 2]
            tx, v = _tc_txv(agg, dis2, prev, coef, emit_v=(k < 5))
            if k == 5 and li == 0:
                acc, v = _tc_mm(acc, tx, W[k], b, dis2=dis2)
            else:
                acc = _tc_mm(acc, tx, W[k], b if k == 5 else None)
            txs.append(tx)
        state = acc

    out = _tc_mm(None, state, Wl, bl)
    return out[:N_T]
